# Initial kernel scaffold; baseline (speedup 1.0000x reference)
import jax
import jax.numpy as jnp
from jax.experimental import pallas as pl


def kernel(x, edge_index, pos, time, batch, w_dist, msg_w1, msg_b1, msg_w2, msg_b2, gate_w, gate_b, time_w1, time_b1, time_w2, time_b2, comb_w1, comb_b1, comb_w2, comb_b2, coord_w1, coord_b1, coord_w2, coord_b2):
    raise NotImplementedError("write your pallas kernel here")


# speedup vs baseline: 2.3725x; 2.3725x over previous
import functools

import jax
import jax.numpy as jnp
import numpy as np
from jax import lax
from jax.experimental import pallas as pl
from jax.experimental.pallas import tpu as pltpu
from jax.experimental.pallas import tpu_sc as plsc

H = 32
MAX_D = 5.0
N = 100000
E = 1600000
NB = 2000
EB = 3200
CH = 512
NCH = E // CH
DCH = 128
NDCH = E // DCH
W = H + 8
NW = 32
HALF = N // 2
ACC_ROWS = 50048
TRASH = 50008


def _silu(x):
    return x * jax.nn.sigmoid(x)


def _stage_a_body(x_ref, b_ref, t_ref, tw1t, tb1, tw2t, tb2, w1at, w1bt, b1,
                  gwt, gb, xm_ref, a_ref, pj_ref):
    t = t_ref[...]
    ss = _silu(jnp.dot(t, tw1t[...], preferred_element_type=jnp.float32) + tb1[...])
    ss = jnp.dot(ss, tw2t[...], preferred_element_type=jnp.float32) + tb2[...]
    b = b_ref[...]
    iota = lax.broadcasted_iota(jnp.int32, (NB, ss.shape[0]), 1).astype(jnp.float32)
    oh = (b == iota).astype(jnp.float32)
    ssn = jnp.dot(oh, ss, preferred_element_type=jnp.float32)
    xm = _silu(x_ref[...] * (1.0 + ssn[:, :H]) + ssn[:, H:])
    xm_ref[...] = xm
    a_ref[...] = jnp.dot(xm, w1at[...], preferred_element_type=jnp.float32)
    bb = jnp.dot(xm, w1bt[...], preferred_element_type=jnp.float32) + b1[...]
    g = jax.nn.sigmoid(jnp.dot(xm, gwt[...], preferred_element_type=jnp.float32) + gb[...])
    pj_ref[...] = jnp.concatenate([bb, g], axis=1)


def _stage_a(x, batchf, time, tw1t, tb1, tw2t, tb2, w1at, w1bt, b1, gwt, gb):
    grid = (N // NB,)
    full = lambda shape: pl.BlockSpec(shape, lambda n: (0, 0))
    row = lambda w: pl.BlockSpec((NB, w), lambda n: (n, 0))
    return pl.pallas_call(
        _stage_a_body,
        grid=grid,
        in_specs=[row(H), row(1), full(time.shape), full(tw1t.shape),
                  full(tb1.shape), full(tw2t.shape), full(tb2.shape),
                  full(w1at.shape), full(w1bt.shape), full(b1.shape),
                  full(gwt.shape), full(gb.shape)],
        out_specs=[row(H), row(H), row(2 * H)],
        out_shape=[jax.ShapeDtypeStruct((N, H), jnp.float32),
                   jax.ShapeDtypeStruct((N, H), jnp.float32),
                   jax.ShapeDtypeStruct((N, 2 * H), jnp.float32)],
    )(x, batchf, time, tw1t, tb1, tw2t, tb2, w1at, w1bt, b1, gwt, gb)


def _gather_body(a_h, pj_h, pos_h, i2_h, j2_h, ai_o, pjo_o, pi_o, pj4_o,
                 idxi, idxj, bA, bPJ, bPI, bPJ4, sem):
    wid = lax.axis_index("s") * 2 + lax.axis_index("c")
    nw = jnp.where(wid < NCH % NW, NCH // NW + 1, NCH // NW)
    base = wid * (NCH // NW) + jnp.minimum(wid, NCH % NW)

    def body(t, carry):
        ch = base + t
        r = ch * (CH // 128)
        eb = ch * CH
        pltpu.sync_copy(i2_h.at[pl.ds(r, CH // 128)], idxi)
        pltpu.sync_copy(j2_h.at[pl.ds(r, CH // 128)], idxj)
        cps = []
        for kk in range(CH // 128):
            sl = pl.ds(kk * 128, 128)
            cps.append(pltpu.async_copy(a_h.at[idxi.at[kk]], bA.at[sl], sem))
            cps.append(pltpu.async_copy(pj_h.at[idxj.at[kk]], bPJ.at[sl], sem))
            cps.append(pltpu.async_copy(pos_h.at[idxi.at[kk]], bPI.at[sl], sem))
            cps.append(pltpu.async_copy(pos_h.at[idxj.at[kk]], bPJ4.at[sl], sem))
        for cp in cps:
            cp.wait()
        pltpu.sync_copy(bA, ai_o.at[pl.ds(eb, CH)])
        pltpu.sync_copy(bPJ, pjo_o.at[pl.ds(eb, CH)])
        pltpu.sync_copy(bPI, pi_o.at[pl.ds(eb, CH)])
        pltpu.sync_copy(bPJ4, pj4_o.at[pl.ds(eb, CH)])
        return carry

    lax.fori_loop(0, nw, body, 0)


def _stage_b(a, pjt, pos4, i2, j2):
    mesh = plsc.VectorSubcoreMesh(core_axis_name="c", subcore_axis_name="s")
    f = pl.kernel(
        _gather_body,
        mesh=mesh,
        out_type=[jax.ShapeDtypeStruct((E, H), jnp.float32),
                  jax.ShapeDtypeStruct((E, 2 * H), jnp.float32),
                  jax.ShapeDtypeStruct((E, 16), jnp.float32),
                  jax.ShapeDtypeStruct((E, 16), jnp.float32)],
        scratch_types=[pltpu.VMEM((CH // 128, 128), jnp.int32),
                       pltpu.VMEM((CH // 128, 128), jnp.int32),
                       pltpu.VMEM((CH, H), jnp.float32),
                       pltpu.VMEM((CH, 2 * H), jnp.float32),
                       pltpu.VMEM((CH, 16), jnp.float32),
                       pltpu.VMEM((CH, 16), jnp.float32),
                       pltpu.SemaphoreType.DMA],
        compiler_params=pltpu.CompilerParams(use_tc_tiling_on_sc=False),
    )
    return f(a, pjt, pos4, i2, j2)


def _stage_c_body(ai_ref, pj_ref, pi_ref, pj4_ref, jf_ref, w2t, b2, wdt, cw1t,
                  cb1, cw2r, cb2, means, betas, mp_ref, jl0_ref, jl1_ref):
    jf = jf_ref[...]
    jl0_ref[...] = jnp.where(jf < HALF, jf, float(TRASH)).astype(jnp.int32)
    jl1_ref[...] = jnp.where(jf >= HALF, jf - HALF, float(TRASH)).astype(jnp.int32)
    a = ai_ref[...]
    pj = pj_ref[...]
    h1 = _silu(a + pj[:, :H])
    g = pj[:, H:]
    h = _silu(jnp.dot(h1, w2t[...], preferred_element_type=jnp.float32) + b2[...])
    pd = pi_ref[...][:, :4] - pj4_ref[...][:, :4]
    d2 = jnp.sum(pd * pd, axis=1, keepdims=True)
    dist = jnp.sqrt(d2)
    d_c = jnp.minimum(dist, MAX_D)
    cutoff = 0.5 * (jnp.cos(d_c * (np.pi / MAX_D)) + 1.0)
    rbf = jnp.exp(-betas[...] * (jnp.exp(-dist) - means[...]) ** 2)
    demb = jnp.dot(cutoff * rbf, wdt[...], preferred_element_type=jnp.float32)
    m = h * demb * g
    cm1 = _silu(jnp.dot(m, cw1t[...], preferred_element_type=jnp.float32) + cb1[...])
    cmsg = jnp.sum(cm1 * cw2r[...], axis=1, keepdims=True) + cb2[...]
    ch3 = (lax.broadcasted_iota(jnp.int32, (EB, 4), 1) == 3).astype(jnp.float32)
    mp_ref[...] = jnp.concatenate([m, pd * cmsg + ch3,
                                   jnp.zeros((EB, 4), jnp.float32)], axis=1)


def _stage_c(ai, pjg, pi4, pj4, jf, w2t, b2, wdt, cw1t, cb1, cw2r, cb2, means, betas):
    grid = (E // EB,)
    full = lambda arr: pl.BlockSpec(arr.shape, lambda n: (0, 0))
    row = lambda w: pl.BlockSpec((EB, w), lambda n: (n, 0))
    return pl.pallas_call(
        _stage_c_body,
        grid=grid,
        in_specs=[row(H), row(2 * H), row(16), row(16), row(1)] +
                 [full(a) for a in (w2t, b2, wdt, cw1t, cb1, cw2r, cb2, means, betas)],
        out_specs=[row(W), row(1), row(1)],
        out_shape=[jax.ShapeDtypeStruct((E, W), jnp.float32),
                   jax.ShapeDtypeStruct((E, 1), jnp.int32),
                   jax.ShapeDtypeStruct((E, 1), jnp.int32)],
    )(ai, pjg, pi4, pj4, jf, w2t, b2, wdt, cw1t, cb1, cw2r, cb2, means, betas)


def _scatter_body(jl_h, mp_h, zmp_h, rows_h, acc_o, idxloc, idxr, bmp, acc):
    cid = lax.axis_index("c")
    sid = lax.axis_index("s")
    scid = cid
    tw = sid
    pltpu.sync_copy(zmp_h, bmp)

    NZC = ACC_ROWS // 128
    nz = jnp.where(tw < NZC % 16, NZC // 16 + 1, NZC // 16)
    zbase = tw * (NZC // 16) + jnp.minimum(tw, NZC % 16)

    def zbody(t, carry):
        pltpu.sync_copy(rows_h.at[pl.ds(zbase + t, 1)], idxr)
        pltpu.sync_copy(bmp, acc.at[idxr.at[0]])
        return carry

    lax.fori_loop(0, nz, zbody, 0)
    plsc.subcore_barrier()

    nt = jnp.where(tw < NDCH % 16, NDCH // 16 + 1, NDCH // 16)
    base = tw * (NDCH // 16) + jnp.minimum(tw, NDCH % 16)

    def body(t, carry):
        ch = base + t
        pltpu.sync_copy(jl_h.at[pl.ds(scid * (E // 128) + ch, 1)], idxloc)
        pltpu.sync_copy(mp_h.at[pl.ds(ch * DCH, DCH)], bmp)
        pltpu.sync_copy(bmp, acc.at[idxloc.at[0]], add=True)
        return carry

    lax.fori_loop(0, nt, body, 0)
    plsc.subcore_barrier()

    NFC = HALF // 128
    nc = jnp.where(tw < NFC % 16, NFC // 16 + 1, NFC // 16)
    cbase = tw * (NFC // 16) + jnp.minimum(tw, NFC % 16)

    def cbody(t, carry):
        c = cbase + t
        pltpu.sync_copy(rows_h.at[pl.ds(c, 1)], idxr)
        pltpu.sync_copy(acc.at[idxr.at[0]], bmp)
        pltpu.sync_copy(bmp, acc_o.at[pl.ds(scid * HALF + c * 128, 128)])
        return carry

    lax.fori_loop(0, nc, cbody, 0)
    TL = HALF - NFC * 128
    pltpu.sync_copy(rows_h.at[pl.ds(NFC, 1)], idxr)
    pltpu.sync_copy(acc.at[idxr.at[0, pl.ds(0, TL)]], bmp.at[pl.ds(0, TL)])
    pltpu.sync_copy(bmp.at[pl.ds(0, TL)],
                    acc_o.at[pl.ds(scid * HALF + NFC * 128, TL)])


def _stage_d(jl, mp, zmp, rows):
    mesh = plsc.VectorSubcoreMesh(core_axis_name="c", subcore_axis_name="s")
    f = pl.kernel(
        _scatter_body,
        mesh=mesh,
        out_type=[jax.ShapeDtypeStruct((N, W), jnp.float32)],
        scratch_types=[pltpu.VMEM((1, 128), jnp.int32),
                       pltpu.VMEM((1, 128), jnp.int32),
                       pltpu.VMEM((DCH, W), jnp.float32),
                       pltpu.VMEM_SHARED((ACC_ROWS, W), jnp.float32)],
        compiler_params=pltpu.CompilerParams(use_tc_tiling_on_sc=False),
    )
    return f(jl, mp, zmp, rows)[0]


def _stage_e_body(xm_ref, acc_ref, pos_ref, cw1at, cw1bt, cb1, cw2t,
                  cb2, xo_ref, po_ref):
    xm = xm_ref[...]
    accv = acc_ref[...]
    mi = accv[:, :H]
    pua = accv[:, H:H + 4]
    c1 = _silu(jnp.dot(xm, cw1at[...], preferred_element_type=jnp.float32) +
               jnp.dot(mi, cw1bt[...], preferred_element_type=jnp.float32) + cb1[...])
    xc = jnp.dot(c1, cw2t[...], preferred_element_type=jnp.float32) + cb2[...]
    xo_ref[...] = _silu(xm + xc)
    cnt = jnp.maximum(pua[:, 3:4], 1.0)
    mask3 = (lax.broadcasted_iota(jnp.int32, (NB, 4), 1) < 3).astype(jnp.float32)
    po_ref[...] = pos_ref[...] + pua * mask3 / cnt


def _stage_e(xm, acc, pos4, cw1at, cw1bt, cb1, cw2t, cb2):
    grid = (N // NB,)
    full = lambda arr: pl.BlockSpec(arr.shape, lambda n: (0, 0))
    row = lambda w: pl.BlockSpec((NB, w), lambda n: (n, 0))
    return pl.pallas_call(
        _stage_e_body,
        grid=grid,
        in_specs=[row(H), row(W), row(4)] +
                 [full(a) for a in (cw1at, cw1bt, cb1, cw2t, cb2)],
        out_specs=[row(H), row(4)],
        out_shape=[jax.ShapeDtypeStruct((N, H), jnp.float32),
                   jax.ShapeDtypeStruct((N, 4), jnp.float32)],
    )(xm, acc, pos4, cw1at, cw1bt, cb1, cw2t, cb2)


def kernel(x, edge_index, pos, time, batch, w_dist, msg_w1, msg_b1, msg_w2,
           msg_b2, gate_w, gate_b, time_w1, time_b1, time_w2, time_b2,
           comb_w1, comb_b1, comb_w2, comb_b2, coord_w1, coord_b1, coord_w2,
           coord_b2):
    batchf = batch.astype(jnp.float32).reshape(N, 1)
    pos4 = jnp.concatenate([pos, jnp.zeros((N, 1), jnp.float32)], axis=1)
    pos16 = jnp.concatenate([pos, jnp.zeros((N, 13), jnp.float32)], axis=1)
    i2 = edge_index[0].reshape(E // 128, 128)
    j2 = edge_index[1].reshape(E // 128, 128)

    xm, a, pjt = _stage_a(
        x, batchf, time,
        time_w1.T, time_b1.reshape(1, -1), time_w2.T, time_b2.reshape(1, -1),
        msg_w1[:, :H].T, msg_w1[:, H:].T, msg_b1.reshape(1, -1),
        gate_w.T, gate_b.reshape(1, -1))

    ai, pjg, pi4, pj4 = _stage_b(a, pjt, pos16, i2, j2)

    start = float(np.exp(-MAX_D))
    means = np.linspace(start, 1.0, H, dtype=np.float32).reshape(1, H)
    betas = np.full((1, H), (2.0 / H * (1.0 - start)) ** (-2), dtype=np.float32)
    jf = edge_index[1].astype(jnp.float32).reshape(E, 1)
    mp, jl0, jl1 = _stage_c(ai, pjg, pi4, pj4, jf,
                            msg_w2.T, msg_b2.reshape(1, -1), w_dist.T,
                            coord_w1.T, coord_b1.reshape(1, -1),
                            coord_w2.reshape(1, -1), coord_b2.reshape(1, 1),
                            jnp.asarray(means), jnp.asarray(betas))
    jl = jnp.concatenate([jl0.reshape(E // 128, 128), jl1.reshape(E // 128, 128)])

    zmp = jnp.zeros((128, W), jnp.float32)
    rows = jnp.arange(ACC_ROWS, dtype=jnp.int32).reshape(ACC_ROWS // 128, 128)
    acc = _stage_d(jl, mp, zmp, rows)

    xo, po4 = _stage_e(xm, acc, pos4,
                       comb_w1[:, :H].T, comb_w1[:, H:].T,
                       comb_b1.reshape(1, -1), comb_w2.T,
                       comb_b2.reshape(1, -1))
    return (xo, po4[:, :3])
